# baseline (speedup 1.0000x reference)
import functools

import jax
import jax.numpy as jnp
from jax import lax
from jax.experimental import pallas as pl
from jax.experimental.pallas import tpu as pltpu
from jax.experimental.pallas import tpu_sc as plsc

N_NODES = 10000
N_EDGES = 320000
IN_DIM = 128
OUT_DIM = 64

NC = 2
NS = 16
NW = NC * NS
CHUNK = 128
N_WIN = N_EDGES // CHUNK
W_BASE = N_WIN // NW
W_EXTRA = N_WIN - W_BASE * NW
W_MAX = W_BASE + 1
NB = 4
NSLOT = 8
N_PAD = 10240
ACC_PER_TILE = N_PAD // NS
OUT_CHUNK = 80


def _mm_tanh_body(fea_ref, w_ref, nf_ref, u1_ref):
    nf = jnp.dot(fea_ref[...], w_ref[...], preferred_element_type=jnp.float32)
    nf_ref[...] = nf
    u1_ref[...] = jnp.tanh(nf)


def _epilogue_body(part_ref, nf_ref, w1_ref, w2_ref, out_ref):
    m = part_ref[0] + part_ref[1]
    h = jnp.maximum(jnp.dot(m, w1_ref[...], preferred_element_type=jnp.float32), 0.0)
    h = jnp.maximum(jnp.dot(h, w2_ref[...], preferred_element_type=jnp.float32), 0.0)
    out_ref[...] = jnp.tanh(nf_ref[...] + h)


def _sc_segment_sum_body(u1_hbm, idx_hbm, out_hbm,
                         src_idx, dst_idx, rows, acc_sh,
                         g_sems, s_sems, rb_sems):
    cid = lax.axis_index("c")
    sid = lax.axis_index("s")
    wid = sid * NC + cid
    acc_base = sid * ACC_PER_TILE

    start = W_BASE * wid + jnp.maximum(0, wid - (NW - W_EXTRA))
    trips = W_BASE + jnp.where(wid >= NW - W_EXTRA, 1, 0)
    pltpu.sync_copy(idx_hbm.at[0, pl.ds(start, W_MAX)], src_idx)
    pltpu.sync_copy(idx_hbm.at[1, pl.ds(start, W_MAX)], dst_idx)

    for b in range(NB):
        pltpu.async_copy(u1_hbm.at[src_idx.at[b]], rows.at[b], g_sems.at[b])

    def _zero(i, carry):
        rows[NSLOT - 1, i // (OUT_DIM // 16),
             pl.ds((i % (OUT_DIM // 16)) * 16, 16)] = jnp.zeros((16,), jnp.float32)
        return carry
    lax.fori_loop(0, CHUNK * (OUT_DIM // 16), _zero, 0)
    for c in range(ACC_PER_TILE // CHUNK):
        pltpu.sync_copy(rows.at[NSLOT - 1],
                        acc_sh.at[pl.ds(acc_base + c * CHUNK, CHUNK)])
    plsc.subcore_barrier()

    def _step(g, j):
        j2 = (j + NB) % NSLOT
        nxt = g + NB

        @pl.when(nxt < trips)
        def _():
            @pl.when(nxt >= NSLOT)
            def _():
                pltpu.make_async_copy(rows.at[j2], acc_sh.at[dst_idx.at[0]],
                                      s_sems.at[j2]).wait()
            pltpu.async_copy(u1_hbm.at[src_idx.at[nxt]], rows.at[j2],
                             g_sems.at[j2])
        pltpu.make_async_copy(u1_hbm.at[src_idx.at[j]], rows.at[j],
                              g_sems.at[j]).wait()
        pltpu.async_copy(rows.at[j], acc_sh.at[dst_idx.at[g]], s_sems.at[j],
                         add=True)

    def _outer(o, carry):
        for j in range(NSLOT):
            _step(o * NSLOT + j, j)
        return carry
    body_windows = (W_BASE // NSLOT) * NSLOT
    lax.fori_loop(0, W_BASE // NSLOT, _outer, 0)
    for t in range(body_windows, W_MAX):

        @pl.when(t < trips)
        def _():
            _step(t, t % NSLOT)
    for j in range(NSLOT):
        pltpu.make_async_copy(rows.at[j], acc_sh.at[dst_idx.at[0]],
                              s_sems.at[j]).wait()

    plsc.subcore_barrier()
    n_out = jnp.minimum(ACC_PER_TILE,
                        jnp.maximum(0, N_NODES - acc_base)) // OUT_CHUNK

    def _rb_pair(p, carry):
        for s in range(2):
            c = 2 * p + s

            @pl.when(c < n_out)
            def _():
                @pl.when(c >= 2)
                def _():
                    prev = acc_base + (c - 2) * OUT_CHUNK
                    pltpu.make_async_copy(
                        rows.at[s, pl.ds(0, OUT_CHUNK)],
                        out_hbm.at[cid, pl.ds(prev, OUT_CHUNK)],
                        rb_sems.at[s]).wait()
                off = acc_base + c * OUT_CHUNK
                pltpu.sync_copy(acc_sh.at[pl.ds(off, OUT_CHUNK)],
                                rows.at[s, pl.ds(0, OUT_CHUNK)])
                pltpu.async_copy(rows.at[s, pl.ds(0, OUT_CHUNK)],
                                 out_hbm.at[cid, pl.ds(off, OUT_CHUNK)],
                                 rb_sems.at[s])
        return carry
    lax.fori_loop(0, (ACC_PER_TILE // OUT_CHUNK + 1) // 2, _rb_pair, 0)
    for s in range(2):
        pltpu.make_async_copy(rows.at[s, pl.ds(0, OUT_CHUNK)],
                              out_hbm.at[cid, pl.ds(acc_base, OUT_CHUNK)],
                              rb_sems.at[s]).wait()


def _sc_segment_sum(u1, idx3d):
    mesh = plsc.VectorSubcoreMesh(core_axis_name="c", subcore_axis_name="s")
    k = functools.partial(
        pl.kernel,
        out_type=jax.ShapeDtypeStruct((NC, N_NODES, OUT_DIM), jnp.float32),
        mesh=mesh,
        compiler_params=pltpu.CompilerParams(use_tc_tiling_on_sc=False),
        scratch_types=[
            pltpu.VMEM((W_MAX, CHUNK), jnp.int32),
            pltpu.VMEM((W_MAX, CHUNK), jnp.int32),
            pltpu.VMEM((NSLOT, CHUNK, OUT_DIM), jnp.float32),
            pltpu.VMEM_SHARED((N_PAD, OUT_DIM), jnp.float32),
            pltpu.SemaphoreType.DMA((NSLOT,)),
            pltpu.SemaphoreType.DMA((NSLOT,)),
            pltpu.SemaphoreType.DMA((2,)),
        ],
    )(_sc_segment_sum_body)
    return k(u1, idx3d)


@jax.jit
def kernel(fea, edge_index, W_lin, W_d1, W_d2):
    idx3d = edge_index.astype(jnp.int32).reshape(2, N_WIN, CHUNK)

    row_block = 2000
    nf, u1 = pl.pallas_call(
        _mm_tanh_body,
        grid=(N_NODES // row_block,),
        in_specs=[
            pl.BlockSpec((row_block, IN_DIM), lambda i: (i, 0)),
            pl.BlockSpec((IN_DIM, OUT_DIM), lambda i: (0, 0)),
        ],
        out_specs=[
            pl.BlockSpec((row_block, OUT_DIM), lambda i: (i, 0)),
            pl.BlockSpec((row_block, OUT_DIM), lambda i: (i, 0)),
        ],
        out_shape=[jax.ShapeDtypeStruct((N_NODES, OUT_DIM), jnp.float32)] * 2,
    )(fea, W_lin)

    partials = _sc_segment_sum(u1, idx3d)

    out = pl.pallas_call(
        _epilogue_body,
        grid=(N_NODES // row_block,),
        in_specs=[
            pl.BlockSpec((NC, row_block, OUT_DIM), lambda i: (0, i, 0)),
            pl.BlockSpec((row_block, OUT_DIM), lambda i: (i, 0)),
            pl.BlockSpec((OUT_DIM, OUT_DIM), lambda i: (0, 0)),
            pl.BlockSpec((OUT_DIM, OUT_DIM), lambda i: (0, 0)),
        ],
        out_specs=pl.BlockSpec((row_block, OUT_DIM), lambda i: (i, 0)),
        out_shape=jax.ShapeDtypeStruct((N_NODES, OUT_DIM), jnp.float32),
    )(partials, nf, W_d1, W_d2)
    return out


# speedup vs baseline: 1.0254x; 1.0254x over previous
import functools

import jax
import jax.numpy as jnp
from jax import lax
from jax.experimental import pallas as pl
from jax.experimental.pallas import tpu as pltpu
from jax.experimental.pallas import tpu_sc as plsc

N_NODES = 10000
N_EDGES = 320000
IN_DIM = 128
OUT_DIM = 64

NC = 2
NS = 16
NW = NC * NS
CHUNK = 128
N_WIN = N_EDGES // CHUNK
W_BASE = N_WIN // NW
W_EXTRA = N_WIN - W_BASE * NW
W_MAX = W_BASE + 1
NB = 4
N_PAD = 10240
ACC_PER_TILE = N_PAD // NS
OUT_CHUNK = 80


def _mm_tanh_body(fea_ref, w_ref, nf_ref, u1_ref):
    nf = jnp.dot(fea_ref[...], w_ref[...], preferred_element_type=jnp.float32)
    nf_ref[...] = nf
    u1_ref[...] = jnp.tanh(nf)


def _epilogue_body(part_ref, nf_ref, w1_ref, w2_ref, out_ref):
    m = part_ref[0] + part_ref[1]
    h = jnp.maximum(jnp.dot(m, w1_ref[...], preferred_element_type=jnp.float32), 0.0)
    h = jnp.maximum(jnp.dot(h, w2_ref[...], preferred_element_type=jnp.float32), 0.0)
    out_ref[...] = jnp.tanh(nf_ref[...] + h)


def _sc_segment_sum_body(u1_hbm, idx_hbm, out_hbm,
                         src_idx, dst_idx, rows, zbuf, acc_sh,
                         g_sems, rb_sems):
    cid = lax.axis_index("c")
    sid = lax.axis_index("s")
    wid = sid * NC + cid
    acc_base = sid * ACC_PER_TILE

    start = W_BASE * wid + jnp.maximum(0, wid - (NW - W_EXTRA))
    trips = W_BASE + jnp.where(wid >= NW - W_EXTRA, 1, 0)
    pltpu.sync_copy(idx_hbm.at[0, pl.ds(start, W_MAX)], src_idx)
    pltpu.sync_copy(idx_hbm.at[1, pl.ds(start, W_MAX)], dst_idx)

    for b in range(NB):
        pltpu.async_copy(u1_hbm.at[src_idx.at[b]], rows.at[b], g_sems.at[b])

    def _zero(i, carry):
        zbuf[i // (OUT_DIM // 16), pl.ds((i % (OUT_DIM // 16)) * 16, 16)] = (
            jnp.zeros((16,), jnp.float32))
        return carry
    lax.fori_loop(0, CHUNK * (OUT_DIM // 16), _zero, 0)
    for c in range(ACC_PER_TILE // CHUNK):
        pltpu.sync_copy(zbuf, acc_sh.at[pl.ds(acc_base + c * CHUNK, CHUNK)])
    plsc.subcore_barrier()

    def _step(g, b):
        pltpu.make_async_copy(u1_hbm.at[src_idx.at[b]], rows.at[b],
                              g_sems.at[b]).wait()
        pltpu.sync_copy(rows.at[b], acc_sh.at[dst_idx.at[g]], add=True)
        nxt = g + NB

        @pl.when(nxt < trips)
        def _():
            pltpu.async_copy(u1_hbm.at[src_idx.at[nxt]], rows.at[b], g_sems.at[b])

    def _outer(o, carry):
        for b in range(NB):
            _step(o * NB + b, b)
        return carry
    body_windows = (W_BASE // NB) * NB
    lax.fori_loop(0, W_BASE // NB, _outer, 0)
    for t in range(body_windows, W_MAX):

        @pl.when(t < trips)
        def _():
            _step(t, t % NB)

    plsc.subcore_barrier()
    n_out = jnp.minimum(ACC_PER_TILE,
                        jnp.maximum(0, N_NODES - acc_base)) // OUT_CHUNK

    def _rb_pair(p, carry):
        for s in range(2):
            c = 2 * p + s

            @pl.when(c < n_out)
            def _():
                @pl.when(c >= 2)
                def _():
                    prev = acc_base + (c - 2) * OUT_CHUNK
                    pltpu.make_async_copy(
                        rows.at[s, pl.ds(0, OUT_CHUNK)],
                        out_hbm.at[cid, pl.ds(prev, OUT_CHUNK)],
                        rb_sems.at[s]).wait()
                off = acc_base + c * OUT_CHUNK
                pltpu.sync_copy(acc_sh.at[pl.ds(off, OUT_CHUNK)],
                                rows.at[s, pl.ds(0, OUT_CHUNK)])
                pltpu.async_copy(rows.at[s, pl.ds(0, OUT_CHUNK)],
                                 out_hbm.at[cid, pl.ds(off, OUT_CHUNK)],
                                 rb_sems.at[s])
        return carry
    lax.fori_loop(0, (ACC_PER_TILE // OUT_CHUNK + 1) // 2, _rb_pair, 0)
    for s in range(2):
        pltpu.make_async_copy(rows.at[s, pl.ds(0, OUT_CHUNK)],
                              out_hbm.at[cid, pl.ds(acc_base, OUT_CHUNK)],
                              rb_sems.at[s]).wait()


def _sc_segment_sum(u1, idx3d):
    mesh = plsc.VectorSubcoreMesh(core_axis_name="c", subcore_axis_name="s")
    k = functools.partial(
        pl.kernel,
        out_type=jax.ShapeDtypeStruct((NC, N_NODES, OUT_DIM), jnp.float32),
        mesh=mesh,
        compiler_params=pltpu.CompilerParams(use_tc_tiling_on_sc=False),
        scratch_types=[
            pltpu.VMEM((W_MAX, CHUNK), jnp.int32),
            pltpu.VMEM((W_MAX, CHUNK), jnp.int32),
            pltpu.VMEM((NB, CHUNK, OUT_DIM), jnp.float32),
            pltpu.VMEM((CHUNK, OUT_DIM), jnp.float32),
            pltpu.VMEM_SHARED((N_PAD, OUT_DIM), jnp.float32),
            pltpu.SemaphoreType.DMA((NB,)),
            pltpu.SemaphoreType.DMA((2,)),
        ],
    )(_sc_segment_sum_body)
    return k(u1, idx3d)


@jax.jit
def kernel(fea, edge_index, W_lin, W_d1, W_d2):
    idx3d = edge_index.astype(jnp.int32).reshape(2, N_WIN, CHUNK)

    row_block = 2000
    nf, u1 = pl.pallas_call(
        _mm_tanh_body,
        grid=(N_NODES // row_block,),
        in_specs=[
            pl.BlockSpec((row_block, IN_DIM), lambda i: (i, 0)),
            pl.BlockSpec((IN_DIM, OUT_DIM), lambda i: (0, 0)),
        ],
        out_specs=[
            pl.BlockSpec((row_block, OUT_DIM), lambda i: (i, 0)),
            pl.BlockSpec((row_block, OUT_DIM), lambda i: (i, 0)),
        ],
        out_shape=[jax.ShapeDtypeStruct((N_NODES, OUT_DIM), jnp.float32)] * 2,
    )(fea, W_lin)

    partials = _sc_segment_sum(u1, idx3d)

    out = pl.pallas_call(
        _epilogue_body,
        grid=(N_NODES // row_block,),
        in_specs=[
            pl.BlockSpec((NC, row_block, OUT_DIM), lambda i: (0, i, 0)),
            pl.BlockSpec((row_block, OUT_DIM), lambda i: (i, 0)),
            pl.BlockSpec((OUT_DIM, OUT_DIM), lambda i: (0, 0)),
            pl.BlockSpec((OUT_DIM, OUT_DIM), lambda i: (0, 0)),
        ],
        out_specs=pl.BlockSpec((row_block, OUT_DIM), lambda i: (i, 0)),
        out_shape=jax.ShapeDtypeStruct((N_NODES, OUT_DIM), jnp.float32),
    )(partials, nf, W_d1, W_d2)
    return out


# speedup vs baseline: 1.0442x; 1.0184x over previous
import functools

import jax
import jax.numpy as jnp
from jax import lax
from jax.experimental import pallas as pl
from jax.experimental.pallas import tpu as pltpu
from jax.experimental.pallas import tpu_sc as plsc

N_NODES = 10000
N_EDGES = 320000
IN_DIM = 128
OUT_DIM = 64

NC = 2
NS = 16
NW = NC * NS
CHUNK = 128
N_WIN = N_EDGES // CHUNK
W_BASE = N_WIN // NW
W_EXTRA = N_WIN - W_BASE * NW
W_MAX = W_BASE + 1
NB = 4
N_PAD = 10240
ACC_PER_TILE = N_PAD // NS
OUT_CHUNK = 80


def _mm_tanh_body(feat_ref, feab_ref, w_ref, nf_ref, u1_ref):
    x = jnp.concatenate([feat_ref[...], feab_ref[...]], axis=1)
    nfp = jnp.dot(x, w_ref[...], preferred_element_type=jnp.float32)
    nf_ref[...] = nfp
    u1_ref[...] = jnp.tanh(nfp)


def _epilogue_body(part_ref, nf_ref, w1_ref, w2_ref, out_ref):
    half = pl.program_id(1)
    m = part_ref[0] + part_ref[1]
    h = jnp.maximum(jnp.dot(m, w1_ref[...], preferred_element_type=jnp.float32), 0.0)
    h = jnp.maximum(jnp.dot(h, w2_ref[...], preferred_element_type=jnp.float32), 0.0)
    o = jnp.tanh(nf_ref[...] + h)
    out_ref[...] = jnp.where(half == 0, o[:, :OUT_DIM], o[:, OUT_DIM:])


def _sc_segment_sum_body(u1_hbm, idx_hbm, out_hbm,
                         src_idx, dst_idx, rows, zbuf, acc_sh,
                         g_sems, rb_sems):
    cid = lax.axis_index("c")
    sid = lax.axis_index("s")
    wid = sid * NC + cid
    acc_base = sid * ACC_PER_TILE

    start = W_BASE * wid + jnp.maximum(0, wid - (NW - W_EXTRA))
    trips = W_BASE + jnp.where(wid >= NW - W_EXTRA, 1, 0)
    pltpu.sync_copy(idx_hbm.at[0, pl.ds(start, W_MAX)], src_idx)
    pltpu.sync_copy(idx_hbm.at[1, pl.ds(start, W_MAX)], dst_idx)

    for b in range(NB):
        pltpu.async_copy(u1_hbm.at[src_idx.at[b]], rows.at[b], g_sems.at[b])

    def _zero(i, carry):
        zbuf[i // (OUT_DIM // 16), pl.ds((i % (OUT_DIM // 16)) * 16, 16)] = (
            jnp.zeros((16,), jnp.float32))
        return carry
    lax.fori_loop(0, CHUNK * (OUT_DIM // 16), _zero, 0)
    for c in range(ACC_PER_TILE // CHUNK):
        pltpu.sync_copy(zbuf, acc_sh.at[pl.ds(acc_base + c * CHUNK, CHUNK)])
    plsc.subcore_barrier()

    def _step(g, b):
        pltpu.make_async_copy(u1_hbm.at[src_idx.at[b]], rows.at[b],
                              g_sems.at[b]).wait()
        pltpu.sync_copy(rows.at[b], acc_sh.at[dst_idx.at[g]], add=True)
        nxt = g + NB

        @pl.when(nxt < trips)
        def _():
            pltpu.async_copy(u1_hbm.at[src_idx.at[nxt]], rows.at[b], g_sems.at[b])

    def _outer(o, carry):
        for b in range(NB):
            _step(o * NB + b, b)
        return carry
    body_windows = (W_BASE // NB) * NB
    lax.fori_loop(0, W_BASE // NB, _outer, 0)
    for t in range(body_windows, W_MAX):

        @pl.when(t < trips)
        def _():
            _step(t, t % NB)

    plsc.subcore_barrier()
    n_out = jnp.minimum(ACC_PER_TILE,
                        jnp.maximum(0, N_NODES - acc_base)) // OUT_CHUNK

    def _rb_pair(p, carry):
        for s in range(2):
            c = 2 * p + s

            @pl.when(c < n_out)
            def _():
                @pl.when(c >= 2)
                def _():
                    prev = acc_base + (c - 2) * OUT_CHUNK
                    pltpu.make_async_copy(
                        rows.at[s, pl.ds(0, OUT_CHUNK)],
                        out_hbm.at[cid, pl.ds(prev, OUT_CHUNK)],
                        rb_sems.at[s]).wait()
                off = acc_base + c * OUT_CHUNK
                pltpu.sync_copy(acc_sh.at[pl.ds(off, OUT_CHUNK)],
                                rows.at[s, pl.ds(0, OUT_CHUNK)])
                pltpu.async_copy(rows.at[s, pl.ds(0, OUT_CHUNK)],
                                 out_hbm.at[cid, pl.ds(off, OUT_CHUNK)],
                                 rb_sems.at[s])
        return carry
    lax.fori_loop(0, (ACC_PER_TILE // OUT_CHUNK + 1) // 2, _rb_pair, 0)
    for s in range(2):
        pltpu.make_async_copy(rows.at[s, pl.ds(0, OUT_CHUNK)],
                              out_hbm.at[cid, pl.ds(acc_base, OUT_CHUNK)],
                              rb_sems.at[s]).wait()


def _sc_segment_sum(u1, idx3d):
    mesh = plsc.VectorSubcoreMesh(core_axis_name="c", subcore_axis_name="s")
    k = functools.partial(
        pl.kernel,
        out_type=jax.ShapeDtypeStruct((NC, N_NODES, OUT_DIM), jnp.float32),
        mesh=mesh,
        compiler_params=pltpu.CompilerParams(use_tc_tiling_on_sc=False),
        scratch_types=[
            pltpu.VMEM((W_MAX, CHUNK), jnp.int32),
            pltpu.VMEM((W_MAX, CHUNK), jnp.int32),
            pltpu.VMEM((NB, CHUNK, OUT_DIM), jnp.float32),
            pltpu.VMEM((CHUNK, OUT_DIM), jnp.float32),
            pltpu.VMEM_SHARED((N_PAD, OUT_DIM), jnp.float32),
            pltpu.SemaphoreType.DMA((NB,)),
            pltpu.SemaphoreType.DMA((2,)),
        ],
    )(_sc_segment_sum_body)
    return k(u1, idx3d)


HALF = N_NODES // 2
PAIR = 2 * OUT_DIM


@jax.jit
def kernel(fea, edge_index, W_lin, W_d1, W_d2):
    ei = edge_index.astype(jnp.int32)
    idx3d = ((ei % HALF) * 2 + ei // HALF).reshape(2, N_WIN, CHUNK)

    z = jnp.zeros((IN_DIM, OUT_DIM), jnp.float32)
    bd_lin = jnp.block([[W_lin, z], [z, W_lin]])
    z = jnp.zeros((OUT_DIM, OUT_DIM), jnp.float32)
    bd_d1 = jnp.block([[W_d1, z], [z, W_d1]])
    bd_d2 = jnp.block([[W_d2, z], [z, W_d2]])

    rb = 1000
    nfp, u1p = pl.pallas_call(
        _mm_tanh_body,
        grid=(HALF // rb,),
        in_specs=[
            pl.BlockSpec((rb, IN_DIM), lambda i: (i, 0)),
            pl.BlockSpec((rb, IN_DIM), lambda i: (i + HALF // rb, 0)),
            pl.BlockSpec((2 * IN_DIM, PAIR), lambda i: (0, 0)),
        ],
        out_specs=[
            pl.BlockSpec((rb, PAIR), lambda i: (i, 0)),
            pl.BlockSpec((rb, PAIR), lambda i: (i, 0)),
        ],
        out_shape=[jax.ShapeDtypeStruct((HALF, PAIR), jnp.float32)] * 2,
    )(fea, fea, bd_lin)

    partials = _sc_segment_sum(u1p.reshape(N_NODES, OUT_DIM), idx3d)

    out = pl.pallas_call(
        _epilogue_body,
        grid=(HALF // rb, 2),
        in_specs=[
            pl.BlockSpec((NC, rb, PAIR), lambda i, h: (0, i, 0)),
            pl.BlockSpec((rb, PAIR), lambda i, h: (i, 0)),
            pl.BlockSpec((PAIR, PAIR), lambda i, h: (0, 0)),
            pl.BlockSpec((PAIR, PAIR), lambda i, h: (0, 0)),
        ],
        out_specs=pl.BlockSpec((rb, OUT_DIM), lambda i, h: (h * (HALF // rb) + i, 0)),
        out_shape=jax.ShapeDtypeStruct((N_NODES, OUT_DIM), jnp.float32),
    )(partials.reshape(NC, HALF, PAIR), nfp, bd_d1, bd_d2)
    return out


# speedup vs baseline: 1.1284x; 1.0806x over previous
import functools

import jax
import jax.numpy as jnp
from jax import lax
from jax.experimental import pallas as pl
from jax.experimental.pallas import tpu as pltpu
from jax.experimental.pallas import tpu_sc as plsc

N_NODES = 10000
N_EDGES = 320000
IN_DIM = 128
OUT_DIM = 64

NC = 2
NS = 16
NW = NC * NS
CHUNK = 128
N_WIN = N_EDGES // CHUNK
W_BASE = N_WIN // NW
W_EXTRA = N_WIN - W_BASE * NW
W_MAX = W_BASE + 1
NB = 4
N_PAD = 10240
ACC_PER_TILE = N_PAD // NS
OUT_CHUNK = 80


def _mm_tanh_body(feat_ref, feab_ref, w_ref, nf_ref, u1_ref):
    x = jnp.concatenate([feat_ref[...], feab_ref[...]], axis=1)
    nfp = jnp.dot(x, w_ref[...], preferred_element_type=jnp.float32)
    nf_ref[...] = nfp
    u1_ref[...] = jnp.tanh(nfp)


def _epilogue_body(part_ref, nf_ref, w1_ref, w2_ref, out_ref):
    half = pl.program_id(1)
    m = part_ref[0] + part_ref[1]
    h = jnp.maximum(jnp.dot(m, w1_ref[...], preferred_element_type=jnp.float32), 0.0)
    h = jnp.maximum(jnp.dot(h, w2_ref[...], preferred_element_type=jnp.float32), 0.0)
    o = jnp.tanh(nf_ref[...] + h)
    out_ref[...] = jnp.where(half == 0, o[:, :OUT_DIM], o[:, OUT_DIM:])


def _sc_segment_sum_body(u1_hbm, idx_hbm, out_hbm,
                         src_idx, dst_idx, rows, zbuf, acc_sh,
                         g_sems, rb_sems):
    cid = lax.axis_index("c")
    sid = lax.axis_index("s")
    wid = sid * NC + cid
    acc_base = sid * ACC_PER_TILE

    start = W_BASE * wid + jnp.maximum(0, wid - (NW - W_EXTRA))
    trips = W_BASE + jnp.where(wid >= NW - W_EXTRA, 1, 0)
    pltpu.sync_copy(idx_hbm.at[0, pl.ds(start, W_MAX)], src_idx)
    pltpu.sync_copy(idx_hbm.at[1, pl.ds(start, W_MAX)], dst_idx)

    for b in range(NB):
        pltpu.async_copy(u1_hbm.at[src_idx.at[b]], rows.at[b], g_sems.at[b])

    def _zero(i, carry):
        zbuf[i // (OUT_DIM // 16), pl.ds((i % (OUT_DIM // 16)) * 16, 16)] = (
            jnp.zeros((16,), jnp.float32))
        return carry
    lax.fori_loop(0, CHUNK * (OUT_DIM // 16), _zero, 0)
    for c in range(ACC_PER_TILE // CHUNK):
        pltpu.sync_copy(zbuf, acc_sh.at[pl.ds(acc_base + c * CHUNK, CHUNK)])
    plsc.subcore_barrier()

    def _step(g, b):
        pltpu.make_async_copy(u1_hbm.at[src_idx.at[b]], rows.at[b],
                              g_sems.at[b]).wait()
        pltpu.sync_copy(rows.at[b], acc_sh.at[dst_idx.at[g]], add=True)
        nxt = g + NB

        @pl.when(nxt < trips)
        def _():
            pltpu.async_copy(u1_hbm.at[src_idx.at[nxt]], rows.at[b], g_sems.at[b])

    def _outer(o, carry):
        for b in range(NB):
            _step(o * NB + b, b)
        return carry
    body_windows = (W_BASE // NB) * NB
    lax.fori_loop(0, W_BASE // NB, _outer, 0)
    for t in range(body_windows, W_MAX):

        @pl.when(t < trips)
        def _():
            _step(t, t % NB)

    plsc.subcore_barrier()
    n_out = jnp.minimum(ACC_PER_TILE,
                        jnp.maximum(0, N_NODES - acc_base)) // OUT_CHUNK

    def _rb_pair(p, carry):
        for s in range(2):
            c = 2 * p + s

            @pl.when(c < n_out)
            def _():
                @pl.when(c >= 2)
                def _():
                    prev = acc_base + (c - 2) * OUT_CHUNK
                    pltpu.make_async_copy(
                        rows.at[s, pl.ds(0, OUT_CHUNK)],
                        out_hbm.at[cid, pl.ds(prev, OUT_CHUNK)],
                        rb_sems.at[s]).wait()
                off = acc_base + c * OUT_CHUNK
                pltpu.sync_copy(acc_sh.at[pl.ds(off, OUT_CHUNK)],
                                rows.at[s, pl.ds(0, OUT_CHUNK)])
                pltpu.async_copy(rows.at[s, pl.ds(0, OUT_CHUNK)],
                                 out_hbm.at[cid, pl.ds(off, OUT_CHUNK)],
                                 rb_sems.at[s])
        return carry
    lax.fori_loop(0, (ACC_PER_TILE // OUT_CHUNK + 1) // 2, _rb_pair, 0)
    for s in range(2):
        pltpu.make_async_copy(rows.at[s, pl.ds(0, OUT_CHUNK)],
                              out_hbm.at[cid, pl.ds(acc_base, OUT_CHUNK)],
                              rb_sems.at[s]).wait()


def _sc_segment_sum(u1, idx3d):
    mesh = plsc.VectorSubcoreMesh(core_axis_name="c", subcore_axis_name="s")
    k = functools.partial(
        pl.kernel,
        out_type=jax.ShapeDtypeStruct((NC, N_NODES, OUT_DIM), jnp.float32),
        mesh=mesh,
        compiler_params=pltpu.CompilerParams(use_tc_tiling_on_sc=False),
        scratch_types=[
            pltpu.VMEM((W_MAX, CHUNK), jnp.int32),
            pltpu.VMEM((W_MAX, CHUNK), jnp.int32),
            pltpu.VMEM((NB, CHUNK, OUT_DIM), jnp.float32),
            pltpu.VMEM((CHUNK, OUT_DIM), jnp.float32),
            pltpu.VMEM_SHARED((N_PAD, OUT_DIM), jnp.float32),
            pltpu.SemaphoreType.DMA((NB,)),
            pltpu.SemaphoreType.DMA((2,)),
        ],
    )(_sc_segment_sum_body)
    return k(u1, idx3d)


HALF = N_NODES // 2
PAIR = 2 * OUT_DIM


@jax.jit
def kernel(fea, edge_index, W_lin, W_d1, W_d2):
    ei = edge_index.astype(jnp.int32)
    idx3d = jnp.where(ei < HALF, 2 * ei,
                      2 * ei - (N_NODES - 1)).reshape(2, N_WIN, CHUNK)

    z = jnp.zeros((IN_DIM, OUT_DIM), jnp.float32)
    bd_lin = jnp.block([[W_lin, z], [z, W_lin]])
    z = jnp.zeros((OUT_DIM, OUT_DIM), jnp.float32)
    bd_d1 = jnp.block([[W_d1, z], [z, W_d1]])
    bd_d2 = jnp.block([[W_d2, z], [z, W_d2]])

    rb = 1000
    nfp, u1p = pl.pallas_call(
        _mm_tanh_body,
        grid=(HALF // rb,),
        in_specs=[
            pl.BlockSpec((rb, IN_DIM), lambda i: (i, 0)),
            pl.BlockSpec((rb, IN_DIM), lambda i: (i + HALF // rb, 0)),
            pl.BlockSpec((2 * IN_DIM, PAIR), lambda i: (0, 0)),
        ],
        out_specs=[
            pl.BlockSpec((rb, PAIR), lambda i: (i, 0)),
            pl.BlockSpec((rb, PAIR), lambda i: (i, 0)),
        ],
        out_shape=[jax.ShapeDtypeStruct((HALF, PAIR), jnp.float32)] * 2,
    )(fea, fea, bd_lin)

    partials = _sc_segment_sum(u1p.reshape(N_NODES, OUT_DIM), idx3d)

    out = pl.pallas_call(
        _epilogue_body,
        grid=(HALF // rb, 2),
        in_specs=[
            pl.BlockSpec((NC, rb, PAIR), lambda i, h: (0, i, 0)),
            pl.BlockSpec((rb, PAIR), lambda i, h: (i, 0)),
            pl.BlockSpec((PAIR, PAIR), lambda i, h: (0, 0)),
            pl.BlockSpec((PAIR, PAIR), lambda i, h: (0, 0)),
        ],
        out_specs=pl.BlockSpec((rb, OUT_DIM), lambda i, h: (h * (HALF // rb) + i, 0)),
        out_shape=jax.ShapeDtypeStruct((N_NODES, OUT_DIM), jnp.float32),
    )(partials.reshape(NC, HALF, PAIR), nfp, bd_d1, bd_d2)
    return out


# speedup vs baseline: 1.1824x; 1.0479x over previous
import functools

import jax
import jax.numpy as jnp
from jax import lax
from jax.experimental import pallas as pl
from jax.experimental.pallas import tpu as pltpu
from jax.experimental.pallas import tpu_sc as plsc

N_NODES = 10000
N_EDGES = 320000
IN_DIM = 128
OUT_DIM = 64

NC = 2
NS = 16
NW = NC * NS
CHUNK = 128
N_WIN = N_EDGES // CHUNK
W_BASE = N_WIN // NW
W_EXTRA = N_WIN - W_BASE * NW
W_MAX = W_BASE + 1
NB = 4
N_PAD = 10240
ACC_PER_TILE = N_PAD // NS
OUT_CHUNK = 80


def _mm_tanh_body(feat_ref, feab_ref, w_ref, nf_ref, u1_ref):
    x = jnp.concatenate([feat_ref[...], feab_ref[...]], axis=1)
    nfp = jnp.dot(x, w_ref[...], preferred_element_type=jnp.float32)
    nf_ref[...] = nfp
    u1_ref[...] = jnp.tanh(nfp)


def _epilogue_body(part_ref, nf_ref, w1_ref, w2_ref, out_ref):
    m = part_ref[0] + part_ref[1]
    h = jnp.maximum(jnp.dot(m, w1_ref[...], preferred_element_type=jnp.float32), 0.0)
    h = jnp.maximum(jnp.dot(h, w2_ref[...], preferred_element_type=jnp.float32), 0.0)
    o = jnp.tanh(nf_ref[...] + h)
    out_ref[0] = o[:, :OUT_DIM]
    out_ref[1] = o[:, OUT_DIM:]


def _sc_segment_sum_body(u1_hbm, idx_hbm, out_hbm,
                         src_idx, dst_idx, rows, zbuf, acc_sh,
                         g_sems, rb_sems):
    cid = lax.axis_index("c")
    sid = lax.axis_index("s")
    wid = sid * NC + cid
    acc_base = sid * ACC_PER_TILE

    start = W_BASE * wid + jnp.maximum(0, wid - (NW - W_EXTRA))
    trips = W_BASE + jnp.where(wid >= NW - W_EXTRA, 1, 0)
    pltpu.sync_copy(idx_hbm.at[0, pl.ds(start, W_MAX)], src_idx)
    pltpu.sync_copy(idx_hbm.at[1, pl.ds(start, W_MAX)], dst_idx)

    for b in range(NB):
        pltpu.async_copy(u1_hbm.at[src_idx.at[b]], rows.at[b], g_sems.at[b])

    def _zero(i, carry):
        zbuf[i // (OUT_DIM // 16), pl.ds((i % (OUT_DIM // 16)) * 16, 16)] = (
            jnp.zeros((16,), jnp.float32))
        return carry
    lax.fori_loop(0, CHUNK * (OUT_DIM // 16), _zero, 0)
    for c in range(ACC_PER_TILE // CHUNK):
        pltpu.sync_copy(zbuf, acc_sh.at[pl.ds(acc_base + c * CHUNK, CHUNK)])
    plsc.subcore_barrier()

    def _step(g, b):
        pltpu.make_async_copy(u1_hbm.at[src_idx.at[b]], rows.at[b],
                              g_sems.at[b]).wait()
        pltpu.sync_copy(rows.at[b], acc_sh.at[dst_idx.at[g]], add=True)
        nxt = g + NB

        @pl.when(nxt < trips)
        def _():
            pltpu.async_copy(u1_hbm.at[src_idx.at[nxt]], rows.at[b], g_sems.at[b])

    def _outer(o, carry):
        for b in range(NB):
            _step(o * NB + b, b)
        return carry
    body_windows = (W_BASE // NB) * NB
    lax.fori_loop(0, W_BASE // NB, _outer, 0)
    for t in range(body_windows, W_MAX):

        @pl.when(t < trips)
        def _():
            _step(t, t % NB)

    plsc.subcore_barrier()
    n_out = jnp.minimum(ACC_PER_TILE,
                        jnp.maximum(0, N_NODES - acc_base)) // OUT_CHUNK

    def _rb_pair(p, carry):
        for s in range(2):
            c = 2 * p + s

            @pl.when(c < n_out)
            def _():
                @pl.when(c >= 2)
                def _():
                    prev = acc_base + (c - 2) * OUT_CHUNK
                    pltpu.make_async_copy(
                        rows.at[s, pl.ds(0, OUT_CHUNK)],
                        out_hbm.at[cid, pl.ds(prev, OUT_CHUNK)],
                        rb_sems.at[s]).wait()
                off = acc_base + c * OUT_CHUNK
                pltpu.sync_copy(acc_sh.at[pl.ds(off, OUT_CHUNK)],
                                rows.at[s, pl.ds(0, OUT_CHUNK)])
                pltpu.async_copy(rows.at[s, pl.ds(0, OUT_CHUNK)],
                                 out_hbm.at[cid, pl.ds(off, OUT_CHUNK)],
                                 rb_sems.at[s])
        return carry
    lax.fori_loop(0, (ACC_PER_TILE // OUT_CHUNK + 1) // 2, _rb_pair, 0)
    for s in range(2):
        pltpu.make_async_copy(rows.at[s, pl.ds(0, OUT_CHUNK)],
                              out_hbm.at[cid, pl.ds(acc_base, OUT_CHUNK)],
                              rb_sems.at[s]).wait()


def _sc_segment_sum(u1, idx3d):
    mesh = plsc.VectorSubcoreMesh(core_axis_name="c", subcore_axis_name="s")
    k = functools.partial(
        pl.kernel,
        out_type=jax.ShapeDtypeStruct((NC, N_NODES, OUT_DIM), jnp.float32),
        mesh=mesh,
        compiler_params=pltpu.CompilerParams(use_tc_tiling_on_sc=False),
        scratch_types=[
            pltpu.VMEM((W_MAX, CHUNK), jnp.int32),
            pltpu.VMEM((W_MAX, CHUNK), jnp.int32),
            pltpu.VMEM((NB, CHUNK, OUT_DIM), jnp.float32),
            pltpu.VMEM((CHUNK, OUT_DIM), jnp.float32),
            pltpu.VMEM_SHARED((N_PAD, OUT_DIM), jnp.float32),
            pltpu.SemaphoreType.DMA((NB,)),
            pltpu.SemaphoreType.DMA((2,)),
        ],
    )(_sc_segment_sum_body)
    return k(u1, idx3d)


HALF = N_NODES // 2
PAIR = 2 * OUT_DIM


@jax.jit
def kernel(fea, edge_index, W_lin, W_d1, W_d2):
    ei = edge_index.astype(jnp.int32)
    idx3d = jnp.where(ei < HALF, 2 * ei,
                      2 * ei - (N_NODES - 1)).reshape(2, N_WIN, CHUNK)

    z = jnp.zeros((IN_DIM, OUT_DIM), jnp.float32)
    bd_lin = jnp.block([[W_lin, z], [z, W_lin]])
    z = jnp.zeros((OUT_DIM, OUT_DIM), jnp.float32)
    bd_d1 = jnp.block([[W_d1, z], [z, W_d1]])
    bd_d2 = jnp.block([[W_d2, z], [z, W_d2]])

    rb = 1000
    nfp, u1p = pl.pallas_call(
        _mm_tanh_body,
        grid=(HALF // rb,),
        in_specs=[
            pl.BlockSpec((rb, IN_DIM), lambda i: (i, 0)),
            pl.BlockSpec((rb, IN_DIM), lambda i: (i + HALF // rb, 0)),
            pl.BlockSpec((2 * IN_DIM, PAIR), lambda i: (0, 0)),
        ],
        out_specs=[
            pl.BlockSpec((rb, PAIR), lambda i: (i, 0)),
            pl.BlockSpec((rb, PAIR), lambda i: (i, 0)),
        ],
        out_shape=[jax.ShapeDtypeStruct((HALF, PAIR), jnp.float32)] * 2,
    )(fea, fea, bd_lin)

    partials = _sc_segment_sum(u1p.reshape(N_NODES, OUT_DIM), idx3d)

    out = pl.pallas_call(
        _epilogue_body,
        grid=(HALF // rb,),
        in_specs=[
            pl.BlockSpec((NC, rb, PAIR), lambda i: (0, i, 0)),
            pl.BlockSpec((rb, PAIR), lambda i: (i, 0)),
            pl.BlockSpec((PAIR, PAIR), lambda i: (0, 0)),
            pl.BlockSpec((PAIR, PAIR), lambda i: (0, 0)),
        ],
        out_specs=pl.BlockSpec((2, rb, OUT_DIM), lambda i: (0, i, 0)),
        out_shape=jax.ShapeDtypeStruct((2, HALF, OUT_DIM), jnp.float32),
    )(partials.reshape(NC, HALF, PAIR), nfp, bd_d1, bd_d2)
    return out.reshape(N_NODES, OUT_DIM)
